# Initial kernel scaffold; baseline (speedup 1.0000x reference)
import functools

import jax
import jax.numpy as jnp
from jax.experimental import pallas as pl


def _nll_block(logits_ref, y_ref, borders_ref, out_ref):
    lg = logits_ref[...]
    yv = y_ref[...]
    b = borders_ref[...]

    m = jnp.max(lg, axis=-1, keepdims=True)
    se = jnp.sum(jnp.exp(lg - m), axis=-1, keepdims=True)
    lse = jnp.log(se) + m

    cnt = jnp.sum((b < yv.reshape(-1, 1)).astype(jnp.int32), axis=-1)
    nb = lg.shape[-1]
    idx = jnp.clip(cnt - 1, 0, nb - 1)

    logw = jnp.log(b[0, 1:] - b[0, :-1])
    onehot = jax.lax.broadcasted_iota(jnp.int32, lg.shape, 1) == idx[:, None]
    picked = jnp.sum(jnp.where(onehot, lg - logw[None, :], 0.0), axis=-1)

    out_ref[...] = (lse[:, 0] - picked).reshape(1, -1)


@jax.jit
def kernel(logits, y, borders):
    bsz, seq, nb = logits.shape
    tokens = bsz * seq
    T = 2048
    grid = tokens // T

    lg2 = logits.reshape(tokens, nb)
    y2 = y.reshape(grid, T)
    b2 = borders.reshape(1, -1)

    out = pl.pallas_call(
        _nll_block,
        grid=(grid,),
        in_specs=[
            pl.BlockSpec((T, nb), lambda i: (i, 0)),
            pl.BlockSpec((1, T), lambda i: (i, 0)),
            pl.BlockSpec((1, borders.shape[0]), lambda i: (0, 0)),
        ],
        out_specs=pl.BlockSpec((1, T), lambda i: (i, 0)),
        out_shape=jax.ShapeDtypeStruct((grid, T), jnp.float32),
    )(lg2, y2, b2)
    return out.reshape(bsz, seq)


# speedup vs baseline: 25.0031x; 25.0031x over previous
import functools

import jax
import jax.numpy as jnp
from jax.experimental import pallas as pl


def _nll_block(logits_ref, y_ref, borders_ref, out_ref):
    lg = logits_ref[...]
    yv = y_ref[...]
    b = borders_ref[...]

    m = jnp.max(lg, axis=-1, keepdims=True)
    se = jnp.sum(jnp.exp(lg - m), axis=-1, keepdims=True)
    lse = jnp.log(se) + m

    cnt = jnp.sum((b < yv.reshape(-1, 1)).astype(jnp.int32), axis=-1)
    nb = lg.shape[-1]
    idx = jnp.clip(cnt - 1, 0, nb - 1)

    logw = jnp.log(b[0, 1:] - b[0, :-1])
    onehot = jax.lax.broadcasted_iota(jnp.int32, lg.shape, 1) == idx[:, None]
    picked = jnp.sum(jnp.where(onehot, lg - logw[None, :], 0.0), axis=-1)

    out_ref[...] = (lse[:, 0] - picked).reshape(1, 1, -1)


@jax.jit
def kernel(logits, y, borders):
    bsz, seq, nb = logits.shape
    tokens = bsz * seq
    T = 2048
    grid = tokens // T

    lg2 = logits.reshape(tokens, nb)
    y2 = y.reshape(grid, 1, T)
    b2 = borders.reshape(1, -1)

    out = pl.pallas_call(
        _nll_block,
        grid=(grid,),
        in_specs=[
            pl.BlockSpec((T, nb), lambda i: (i, 0)),
            pl.BlockSpec((1, 1, T), lambda i: (i, 0, 0)),
            pl.BlockSpec((1, borders.shape[0]), lambda i: (0, 0)),
        ],
        out_specs=pl.BlockSpec((1, 1, T), lambda i: (i, 0, 0)),
        out_shape=jax.ShapeDtypeStruct((grid, 1, T), jnp.float32),
    )(lg2, y2, b2)
    return out.reshape(bsz, seq)
